# Initial kernel scaffold; baseline (speedup 1.0000x reference)
import jax
import jax.numpy as jnp
from jax.experimental import pallas as pl


def kernel(x, edge_index):
    raise NotImplementedError("write your pallas kernel here")


# speedup vs baseline: 1002.3098x; 1002.3098x over previous
import functools

import jax
import jax.numpy as jnp
from jax import lax
from jax.experimental import pallas as pl
from jax.experimental.pallas import tpu as pltpu
from jax.experimental.pallas import tpu_sc as plsc

NUM_ITERS = 3
NC = 2
NS = 16
NW = NC * NS
LANES = 16
MASK16 = 0xFFFF


def _argmax_body(x_ref, o_ref):
    o_ref[...] = lax.argmax(x_ref[...], 1, jnp.int32)[:, None]


def _scatter_body(np_, epw, e_pad, labels_hbm, col_hbm, row_hbm, wl_hbm,
                  out_hbm, lab_v, acc_v, col_v, row_v, wl_v):
    i32 = jnp.int32
    wid = lax.axis_index("s") * i32(NC) + lax.axis_index("c")
    base = wid * i32(epw)
    pltpu.sync_copy(labels_hbm, lab_v)
    pltpu.sync_copy(col_hbm.at[pl.ds(base, epw)], col_v)
    pltpu.sync_copy(row_hbm.at[pl.ds(base, epw)], row_v)
    for l in range(4):
        pltpu.sync_copy(wl_hbm.at[pl.ds(i32(l * e_pad) + base, epw)],
                        wl_v.at[pl.ds(i32(l * epw), epw)])

    zeros = jnp.zeros((LANES,), jnp.int32)

    def zero_body(i, carry):
        acc_v[pl.ds(i * i32(LANES), LANES)] = zeros
        return carry

    lax.fori_loop(i32(0), i32((4 * np_) // LANES), zero_body, i32(0))

    def edge_body(i, carry):
        sl = pl.ds(i * i32(LANES), LANES)
        c = col_v[sl]
        r = row_v[sl]
        a = [plsc.load_gather(lab_v, [c if l == 0 else c + i32(l * np_)])
             for l in range(4)]
        b = [wl_v[pl.ds(i32(l * epw) + i * i32(LANES), LANES)]
             for l in range(4)]
        p = [zeros, zeros, zeros, zeros]
        for ia in range(4):
            for ib in range(4 - ia):
                k = ia + ib
                prod = a[ia] * b[ib]
                p[k] = p[k] + (prod & MASK16)
                if k < 3:
                    p[k + 1] = p[k + 1] + ((prod >> 16) & MASK16)
        t = p[0]
        q0 = t & MASK16
        t = p[1] + (t >> 16)
        q1 = t & MASK16
        t = p[2] + (t >> 16)
        q2 = t & MASK16
        t = p[3] + (t >> 16)
        q3 = t & MASK16
        for l, q in enumerate((q0, q1, q2, q3)):
            plsc.addupdate_scatter(acc_v, [r if l == 0 else r + i32(l * np_)], q)
        return carry

    lax.fori_loop(i32(0), i32(epw // LANES), edge_body, i32(0))

    def norm_body(i, carry):
        sls = [pl.ds(i32(l * np_) + i * i32(LANES), LANES) for l in range(4)]
        t = acc_v[sls[0]]
        acc_v[sls[0]] = t & MASK16
        t = acc_v[sls[1]] + (t >> 16)
        acc_v[sls[1]] = t & MASK16
        t = acc_v[sls[2]] + (t >> 16)
        acc_v[sls[2]] = t & MASK16
        t = acc_v[sls[3]] + (t >> 16)
        acc_v[sls[3]] = t & MASK16
        return carry

    lax.fori_loop(i32(0), i32(np_ // LANES), norm_body, i32(0))

    for l in range(4):
        pltpu.sync_copy(acc_v.at[pl.ds(i32(l * np_), np_)],
                        out_hbm.at[i32(l * NW) + wid])


def _reduce_body(p_ref, limb_ref, pack_ref):
    v = p_ref[...]
    s = [jnp.sum(v[l * NW:(l + 1) * NW, :], axis=0, keepdims=True,
                 dtype=jnp.int32)
         for l in range(4)]
    t = s[0]
    q0 = t & MASK16
    t = s[1] + (t >> 16)
    q1 = t & MASK16
    t = s[2] + (t >> 16)
    q2 = t & MASK16
    t = s[3] + (t >> 16)
    q3 = t & MASK16
    limb_ref[...] = jnp.concatenate([q0, q1, q2, q3], axis=0)
    lo = q0 | (q1 << 16)
    hi = q2 | (q3 << 16)
    pack_ref[...] = jnp.concatenate([lo, hi], axis=0)


def _pack_to_i64(pack, n):
    lo = pack[0, :n]
    hi = pack[1, :n]
    return (hi.astype(jnp.int64) << 32) | lo.astype(jnp.uint32).astype(jnp.int64)


def kernel(x, edge_index):
    n_nodes, _ = x.shape
    n_edges = edge_index.shape[1]

    np_ = ((n_nodes + 1 + 127) // 128) * 128
    epw = ((n_edges + NW * LANES - 1) // (NW * LANES)) * LANES
    e_pad = NW * epw
    g = e_pad // LANES

    row = edge_index[0].astype(jnp.int32)
    col = edge_index[1].astype(jnp.int32)

    perm = jnp.argsort(row, stable=True)
    row_s = row[perm]
    col_s = col[perm]
    idx = jnp.arange(n_edges, dtype=jnp.int32)
    is_last = jnp.concatenate([row_s[1:] != row_s[:-1],
                               jnp.ones((1,), dtype=bool)])
    endv = jnp.where(is_last, idx, jnp.int32(n_edges))
    end_idx = jnp.flip(lax.cummin(jnp.flip(endv)))
    k_cnt = (end_idx - idx).astype(jnp.uint64)

    w = jnp.ones((n_edges,), jnp.uint64)
    basep = jnp.uint64(31)
    for b in range(18):
        bit = (k_cnt >> jnp.uint64(b)) & jnp.uint64(1)
        w = jnp.where(bit == jnp.uint64(1), w * basep, w)
        basep = basep * basep
    wl = jnp.stack([((w >> jnp.uint64(16 * l)) & jnp.uint64(MASK16)).astype(jnp.int32)
                    for l in range(4)])

    pad = e_pad - n_edges
    row_p = jnp.concatenate([row_s, jnp.full((pad,), n_nodes, jnp.int32)])
    col_p = jnp.concatenate([col_s, jnp.zeros((pad,), jnp.int32)])
    wl_p = jnp.concatenate([wl, jnp.zeros((4, pad), jnp.int32)], axis=1)
    deal = lambda v: v.reshape(LANES, g).T.reshape(-1)
    row_r = deal(row_p)
    col_r = deal(col_p)
    wl_r = wl_p.reshape(4, LANES, g).transpose(0, 2, 1).reshape(4 * e_pad)

    x_pad = jnp.pad(x, ((0, np_ - n_nodes), (0, 0)))
    labels0 = pl.pallas_call(
        _argmax_body,
        out_shape=jax.ShapeDtypeStruct((np_, 1), jnp.int32),
    )(x_pad)[:, 0]
    labels_flat = jnp.zeros((4, np_), jnp.int32).at[0].set(labels0).reshape(-1)

    mesh = plsc.VectorSubcoreMesh(core_axis_name="c", subcore_axis_name="s",
                                  num_cores=NC, num_subcores=NS)
    scatter_k = functools.partial(
        pl.kernel,
        out_type=jax.ShapeDtypeStruct((4 * NW, np_), jnp.int32),
        mesh=mesh,
        compiler_params=pltpu.CompilerParams(needs_layout_passes=False),
        scratch_types=[
            pltpu.VMEM((4 * np_,), jnp.int32),
            pltpu.VMEM((4 * np_,), jnp.int32),
            pltpu.VMEM((epw,), jnp.int32),
            pltpu.VMEM((epw,), jnp.int32),
            pltpu.VMEM((4 * epw,), jnp.int32),
        ],
    )(functools.partial(_scatter_body, np_, epw, e_pad))

    reduce_k = pl.pallas_call(
        _reduce_body,
        out_shape=[jax.ShapeDtypeStruct((4, np_), jnp.int32),
                   jax.ShapeDtypeStruct((2, np_), jnp.int32)],
    )

    history = [labels0[:n_nodes].astype(jnp.int64)]
    for _ in range(NUM_ITERS):
        partials = scatter_k(labels_flat, col_r, row_r, wl_r)
        limbs, pack = reduce_k(partials)
        labels_flat = limbs.reshape(-1)
        history.append(_pack_to_i64(pack, n_nodes))

    return history[-1], jnp.stack(history)


# speedup vs baseline: 1385.5958x; 1.3824x over previous
import functools

import jax
import jax.numpy as jnp
from jax import lax
from jax.experimental import pallas as pl
from jax.experimental.pallas import tpu as pltpu
from jax.experimental.pallas import tpu_sc as plsc

NUM_ITERS = 3
NC = 2
NS = 16
NW = NC * NS
LANES = 16
MASK16 = 0xFFFF


def _argmax_body(x_ref, o_ref):
    o_ref[...] = lax.argmax(x_ref[...], 1, jnp.int32)[:, None]


def _edge_step(np_, epw, lab_v, acc_v, col_v, row_v, wl_v, i):
    i32 = jnp.int32
    sl = pl.ds(i * i32(LANES), LANES)
    c = col_v[sl]
    r = row_v[sl]
    g0 = plsc.load_gather(lab_v, [c])
    g1 = plsc.load_gather(lab_v, [c + i32(np_)])
    a0 = g0 & MASK16
    a1 = (g0 >> 16) & MASK16
    a2 = g1 & MASK16
    a3 = (g1 >> 16) & MASK16
    w01 = wl_v[pl.ds(i * i32(LANES), LANES)]
    w23 = wl_v[pl.ds(i32(epw) + i * i32(LANES), LANES)]
    b0 = w01 & MASK16
    b1 = (w01 >> 16) & MASK16
    b2 = w23 & MASK16
    b3 = (w23 >> 16) & MASK16

    m00 = a0 * b0
    m01 = a0 * b1
    m10 = a1 * b0
    m02 = a0 * b2
    m11 = a1 * b1
    m20 = a2 * b0
    t = m00 & MASK16
    q0 = t
    p1 = ((m00 >> 16) & MASK16) + (m01 & MASK16) + (m10 & MASK16)
    t = p1 + (t >> 16)
    q1 = t & MASK16
    p2 = (((m01 >> 16) & MASK16) + ((m10 >> 16) & MASK16)
          + (m02 & MASK16) + (m11 & MASK16) + (m20 & MASK16))
    t = p2 + (t >> 16)
    q2 = t & MASK16
    p3 = (a0 * b3 + a1 * b2 + a2 * b1 + a3 * b0
          + (m02 >> 16) + (m11 >> 16) + (m20 >> 16))
    t = p3 + (t >> 16)
    q3 = t & MASK16

    plsc.addupdate_scatter(acc_v, [r], q0)
    plsc.addupdate_scatter(acc_v, [r + i32(np_)], q1)
    plsc.addupdate_scatter(acc_v, [r + i32(2 * np_)], q2)
    plsc.addupdate_scatter(acc_v, [r + i32(3 * np_)], q3)


def _scatter_body(np_, epw, e_pad, labels_hbm, col_hbm, row_hbm, wl_hbm,
                  out_hbm, lab_v, acc_v, col_v, row_v, wl_v):
    i32 = jnp.int32
    wid = lax.axis_index("s") * i32(NC) + lax.axis_index("c")
    base = wid * i32(epw)
    pltpu.sync_copy(labels_hbm, lab_v)
    pltpu.sync_copy(col_hbm.at[pl.ds(base, epw)], col_v)
    pltpu.sync_copy(row_hbm.at[pl.ds(base, epw)], row_v)
    for l in range(2):
        pltpu.sync_copy(wl_hbm.at[pl.ds(i32(l * e_pad) + base, epw)],
                        wl_v.at[pl.ds(i32(l * epw), epw)])

    zeros = jnp.zeros((LANES,), jnp.int32)
    ZU = 8

    def zero_body(i, carry):
        for u in range(ZU):
            acc_v[pl.ds(i * i32(ZU * LANES) + i32(u * LANES), LANES)] = zeros
        return carry

    lax.fori_loop(i32(0), i32((4 * np_) // (ZU * LANES)), zero_body, i32(0))

    EU = 4
    step = functools.partial(_edge_step, np_, epw, lab_v, acc_v, col_v, row_v,
                             wl_v)

    def edge_body(i, carry):
        for u in range(EU):
            step(i * i32(EU) + i32(u))
        return carry

    lax.fori_loop(i32(0), i32(epw // (EU * LANES)), edge_body, i32(0))

    def norm_body(i, carry):
        for u in range(2):
            off = i * i32(2 * LANES) + i32(u * LANES)
            sls = [pl.ds(i32(l * np_) + off, LANES) for l in range(4)]
            t = acc_v[sls[0]]
            acc_v[sls[0]] = t & MASK16
            t = acc_v[sls[1]] + (t >> 16)
            acc_v[sls[1]] = t & MASK16
            t = acc_v[sls[2]] + (t >> 16)
            acc_v[sls[2]] = t & MASK16
            t = acc_v[sls[3]] + (t >> 16)
            acc_v[sls[3]] = t & MASK16
        return carry

    lax.fori_loop(i32(0), i32(np_ // (2 * LANES)), norm_body, i32(0))

    for l in range(4):
        pltpu.sync_copy(acc_v.at[pl.ds(i32(l * np_), np_)],
                        out_hbm.at[i32(l * NW) + wid])


def _reduce_body(p_ref, pack_ref):
    v = p_ref[...]
    s = [jnp.sum(v[l * NW:(l + 1) * NW, :], axis=0, keepdims=True,
                 dtype=jnp.int32)
         for l in range(4)]
    t = s[0]
    q0 = t & MASK16
    t = s[1] + (t >> 16)
    q1 = t & MASK16
    t = s[2] + (t >> 16)
    q2 = t & MASK16
    t = s[3] + (t >> 16)
    q3 = t & MASK16
    lo = q0 | (q1 << 16)
    hi = q2 | (q3 << 16)
    pack_ref[...] = jnp.concatenate([lo, hi], axis=0)


def _pack_to_i64(pack, n):
    lo = pack[0, :n]
    hi = pack[1, :n]
    return (hi.astype(jnp.int64) << 32) | lo.astype(jnp.uint32).astype(jnp.int64)


def kernel(x, edge_index):
    n_nodes, _ = x.shape
    n_edges = edge_index.shape[1]

    np_ = ((n_nodes + 1 + 127) // 128) * 128
    epw = ((n_edges + NW * 4 * LANES - 1) // (NW * 4 * LANES)) * 4 * LANES
    e_pad = NW * epw
    g = e_pad // LANES

    row = edge_index[0].astype(jnp.int32)
    col = edge_index[1].astype(jnp.int32)

    idx = jnp.arange(n_edges, dtype=jnp.int32)
    key = (row.astype(jnp.uint32) << 18) | idx.astype(jnp.uint32)
    key_s = jnp.sort(key)
    row_s = (key_s >> 18).astype(jnp.int32)
    perm = (key_s & jnp.uint32((1 << 18) - 1)).astype(jnp.int32)
    col_s = col[perm]
    is_last = jnp.concatenate([row_s[1:] != row_s[:-1],
                               jnp.ones((1,), dtype=bool)])
    endv = jnp.where(is_last, idx, jnp.int32(n_edges))
    end_idx = jnp.flip(lax.cummin(jnp.flip(endv)))
    k_cnt = (end_idx - idx).astype(jnp.uint64)

    w = jnp.ones((n_edges,), jnp.uint64)
    basep = jnp.uint64(31)
    for b in range(18):
        bit = (k_cnt >> jnp.uint64(b)) & jnp.uint64(1)
        w = jnp.where(bit == jnp.uint64(1), w * basep, w)
        basep = basep * basep
    wp = jnp.stack(
        [lax.bitcast_convert_type((w & jnp.uint64(0xFFFFFFFF)).astype(jnp.uint32),
                                  jnp.int32),
         lax.bitcast_convert_type((w >> jnp.uint64(32)).astype(jnp.uint32),
                                  jnp.int32)])

    pad = e_pad - n_edges
    row_p = jnp.concatenate([row_s, jnp.full((pad,), n_nodes, jnp.int32)])
    col_p = jnp.concatenate([col_s, jnp.zeros((pad,), jnp.int32)])
    wp_p = jnp.concatenate([wp, jnp.zeros((2, pad), jnp.int32)], axis=1)
    deal = lambda v: v.reshape(LANES, g).T.reshape(-1)
    row_r = deal(row_p)
    col_r = deal(col_p)
    wp_r = wp_p.reshape(2, LANES, g).transpose(0, 2, 1).reshape(2 * e_pad)

    x_pad = jnp.pad(x, ((0, np_ - n_nodes), (0, 0)))
    labels0 = pl.pallas_call(
        _argmax_body,
        out_shape=jax.ShapeDtypeStruct((np_, 1), jnp.int32),
    )(x_pad)[:, 0]
    labels_flat = jnp.zeros((2, np_), jnp.int32).at[0].set(labels0).reshape(-1)

    mesh = plsc.VectorSubcoreMesh(core_axis_name="c", subcore_axis_name="s",
                                  num_cores=NC, num_subcores=NS)
    scatter_k = functools.partial(
        pl.kernel,
        out_type=jax.ShapeDtypeStruct((4 * NW, np_), jnp.int32),
        mesh=mesh,
        compiler_params=pltpu.CompilerParams(needs_layout_passes=False),
        scratch_types=[
            pltpu.VMEM((2 * np_,), jnp.int32),
            pltpu.VMEM((4 * np_,), jnp.int32),
            pltpu.VMEM((epw,), jnp.int32),
            pltpu.VMEM((epw,), jnp.int32),
            pltpu.VMEM((2 * epw,), jnp.int32),
        ],
    )(functools.partial(_scatter_body, np_, epw, e_pad))

    reduce_k = pl.pallas_call(
        _reduce_body,
        out_shape=jax.ShapeDtypeStruct((2, np_), jnp.int32),
    )

    history = [labels0[:n_nodes].astype(jnp.int64)]
    for _ in range(NUM_ITERS):
        partials = scatter_k(labels_flat, col_r, row_r, wp_r)
        pack = reduce_k(partials)
        labels_flat = pack.reshape(-1)
        history.append(_pack_to_i64(pack, n_nodes))

    return history[-1], jnp.stack(history)
